# Initial kernel scaffold; baseline (speedup 1.0000x reference)
import functools

import jax
import jax.numpy as jnp
from jax import lax
from jax.experimental import pallas as pl
from jax.experimental.pallas import tpu as pltpu
from jax.experimental.pallas import tpu_sc as plsc

N = 10000
E = 320000
D = 128
A = 8
H = 32

NC = 2
NS = 16
NW = NC * NS
CHUNK = 80
EDGES_PER_W = E // NW
NCHUNK = EDGES_PER_W // CHUNK
NPAD = 10240
DEGW = 8
ROWS_PER_S = N // NS
ZR = 25

_mesh = plsc.VectorSubcoreMesh(core_axis_name="c", subcore_axis_name="s")


def _deg_body(dst_hbm, degp_hbm, idx_v, ones_v, zer_v, deg_sh):
    cid = lax.axis_index("c")
    sid = lax.axis_index("s")
    for r in range(CHUNK):
        for k in range(DEGW // 8):
            ones_v[r, pl.ds(k * 16, 16)] = jnp.full((16,), 1.0, jnp.float32)
    for r in range(640):
        for k in range(DEGW // 8):
            zer_v[r, pl.ds(k * 16, 16)] = jnp.zeros((16,), jnp.float32)
    pltpu.sync_copy(zer_v, deg_sh.at[pl.ds(sid * 640, 640)])
    plsc.subcore_barrier()
    base = (cid * NS + sid) * EDGES_PER_W

    def body(i, carry):
        off = base + i * CHUNK
        pltpu.sync_copy(dst_hbm.at[pl.ds(off, CHUNK)], idx_v)
        pltpu.sync_copy(ones_v, deg_sh.at[idx_v], add=True)
        return carry

    lax.fori_loop(0, NCHUNK, body, 0)
    plsc.subcore_barrier()
    pltpu.sync_copy(deg_sh.at[pl.ds(sid * 640, 640)],
                    degp_hbm.at[pl.ds(cid * NPAD + sid * 640, 640)])


def _agg_body(g_hbm, src_hbm, dst_hbm, accp_hbm, src_v, dst_v, rows_v, zer_v,
              acc_sh, sem):
    cid = lax.axis_index("c")
    sid = lax.axis_index("s")
    for r in range(ZR):
        for k in range(D // 16):
            zer_v[r, pl.ds(k * 16, 16)] = jnp.zeros((16,), jnp.float32)
    for t in range(ROWS_PER_S // ZR):
        pltpu.sync_copy(zer_v, acc_sh.at[pl.ds(sid * ROWS_PER_S + t * ZR, ZR)])
    plsc.subcore_barrier()
    base = (cid * NS + sid) * EDGES_PER_W

    def body(i, carry):
        off = base + i * CHUNK
        pltpu.sync_copy(src_hbm.at[pl.ds(off, CHUNK)], src_v)
        pltpu.sync_copy(dst_hbm.at[pl.ds(off, CHUNK)], dst_v)
        pltpu.async_copy(g_hbm.at[src_v], rows_v, sem).wait()
        pltpu.sync_copy(rows_v, acc_sh.at[dst_v], add=True)
        return carry

    lax.fori_loop(0, NCHUNK, body, 0)
    plsc.subcore_barrier()
    pltpu.sync_copy(acc_sh.at[pl.ds(sid * ROWS_PER_S, ROWS_PER_S)],
                    accp_hbm.at[pl.ds(cid * N + sid * ROWS_PER_S, ROWS_PER_S)])


_deg_call = functools.partial(
    pl.kernel,
    out_type=jax.ShapeDtypeStruct((2 * NPAD, DEGW), jnp.float32),
    mesh=_mesh,
    scratch_types=[
        pltpu.VMEM((CHUNK,), jnp.int32),
        pltpu.VMEM((CHUNK, DEGW), jnp.float32),
        pltpu.VMEM((640, DEGW), jnp.float32),
        pltpu.VMEM_SHARED((NPAD, DEGW), jnp.float32),
    ],
)(_deg_body)

_agg_call = functools.partial(
    pl.kernel,
    out_type=jax.ShapeDtypeStruct((2 * N, D), jnp.float32),
    mesh=_mesh,
    scratch_types=[
        pltpu.VMEM((CHUNK,), jnp.int32),
        pltpu.VMEM((CHUNK,), jnp.int32),
        pltpu.VMEM((CHUNK, D), jnp.float32),
        pltpu.VMEM((ZR, D), jnp.float32),
        pltpu.VMEM_SHARED((N, D), jnp.float32),
        pltpu.SemaphoreType.DMA,
    ],
)(_agg_body)


def _scale_body(state_ref, w_ref, degp_ref, g_ref, dinv_ref):
    h = jnp.dot(state_ref[...], w_ref[...], preferred_element_type=jnp.float32)
    deg = degp_ref[0][:N, 0:1] + degp_ref[1][:N, 0:1] + 1.0
    dinv = lax.rsqrt(deg)
    g_ref[...] = h * dinv
    dinv_ref[...] = dinv


def _head_body(accp_ref, g_ref, dinv_ref, state_ref, bconv_ref, w1_ref, b1_ref,
               w2_ref, b2_ref, w3_ref, b3_ref, out_ref):
    acc = accp_ref[0] + accp_ref[1] + g_ref[...]
    x = acc * dinv_ref[...] + bconv_ref[...]
    x = jnp.maximum(x, 0.0) + state_ref[...]
    y = jnp.dot(x, w1_ref[...], preferred_element_type=jnp.float32) + b1_ref[...]
    y = jnp.where(y > 0, y, 0.01 * y)
    z = jnp.dot(y, w2_ref[...], preferred_element_type=jnp.float32) + b2_ref[...]
    z = jnp.where(z > 0, z, 0.01 * z)
    c = jnp.dot(z, w3_ref[...], preferred_element_type=jnp.float32) + b3_ref[...]
    c = jnp.log1p(jnp.exp(-jnp.abs(c))) + jnp.maximum(c, 0.0)
    out_ref[...] = c / (jnp.sum(c) + 1e-20)


def kernel(state, edge_index, W_conv, b_conv, W1, b1, W2, b2, W3, b3,
           deterministic):
    src = edge_index[0]
    dst = edge_index[1]

    degp = _deg_call(dst)
    degp3 = degp.reshape(2, NPAD, DEGW)

    g, dinv = pl.pallas_call(
        _scale_body,
        out_shape=[
            jax.ShapeDtypeStruct((N, D), jnp.float32),
            jax.ShapeDtypeStruct((N, 1), jnp.float32),
        ],
    )(state, W_conv, degp3)

    accp = _agg_call(g, src, dst)
    accp3 = accp.reshape(2, N, D)

    c = pl.pallas_call(
        _head_body,
        out_shape=jax.ShapeDtypeStruct((N, 1), jnp.float32),
    )(accp3, g, dinv, state, b_conv.reshape(1, D), W1, b1.reshape(1, H),
      W2, b2.reshape(1, H), W3, b3.reshape(1, 1))

    return c.reshape(N // A, A)


# speedup vs baseline: 15.4682x; 15.4682x over previous
import functools

import jax
import jax.numpy as jnp
from jax import lax
from jax.experimental import pallas as pl
from jax.experimental.pallas import tpu as pltpu
from jax.experimental.pallas import tpu_sc as plsc

N = 10000
E = 320000
D = 128
A = 8
H = 32

NC = 2
NS = 16
NW = NC * NS
CHUNK = 80
EDGES_PER_W = E // NW
NCHUNK = EDGES_PER_W // CHUNK
NPAD = 10240
DEGW = 16
ROWS_PER_S = NPAD // NS
ZR = 32

_mesh = plsc.VectorSubcoreMesh(core_axis_name="c", subcore_axis_name="s")


def _deg_body(dst_hbm, degp_hbm, idx_v, ones_v, zer_v, deg_sh):
    cid = lax.axis_index("c")
    sid = lax.axis_index("s")
    for r in range(CHUNK):
        ones_v[r, :] = jnp.full((16,), 1.0, jnp.float32)
    for r in range(ZR):
        zer_v[r, :] = jnp.zeros((16,), jnp.float32)
    for t in range(ROWS_PER_S // ZR):
        pltpu.sync_copy(zer_v, deg_sh.at[pl.ds(sid * ROWS_PER_S + t * ZR, ZR)])
    plsc.subcore_barrier()
    base = (cid * NS + sid) * EDGES_PER_W

    def body(i, carry):
        off = base + i * CHUNK
        pltpu.sync_copy(dst_hbm.at[pl.ds(off, CHUNK)], idx_v)
        pltpu.sync_copy(ones_v, deg_sh.at[idx_v], add=True)
        return carry

    lax.fori_loop(0, NCHUNK, body, 0)
    plsc.subcore_barrier()
    pltpu.sync_copy(deg_sh.at[pl.ds(sid * ROWS_PER_S, ROWS_PER_S)],
                    degp_hbm.at[pl.ds(cid * NPAD + sid * ROWS_PER_S,
                                      ROWS_PER_S)])


def _agg_body(g_hbm, src_hbm, dst_hbm, accp_hbm, src_v, dst_v, rows_v, zer_v,
              acc_sh, sem):
    cid = lax.axis_index("c")
    sid = lax.axis_index("s")
    for r in range(ZR):
        for k in range(D // 16):
            zer_v[r, pl.ds(k * 16, 16)] = jnp.zeros((16,), jnp.float32)
    for t in range(ROWS_PER_S // ZR):
        pltpu.sync_copy(zer_v, acc_sh.at[pl.ds(sid * ROWS_PER_S + t * ZR, ZR)])
    plsc.subcore_barrier()
    base = (cid * NS + sid) * EDGES_PER_W

    def body(i, carry):
        off = base + i * CHUNK
        pltpu.sync_copy(src_hbm.at[pl.ds(off, CHUNK)], src_v)
        pltpu.sync_copy(dst_hbm.at[pl.ds(off, CHUNK)], dst_v)
        pltpu.async_copy(g_hbm.at[src_v], rows_v, sem).wait()
        pltpu.sync_copy(rows_v, acc_sh.at[dst_v], add=True)
        return carry

    lax.fori_loop(0, NCHUNK, body, 0)
    plsc.subcore_barrier()
    pltpu.sync_copy(acc_sh.at[pl.ds(sid * ROWS_PER_S, ROWS_PER_S)],
                    accp_hbm.at[pl.ds(cid * NPAD + sid * ROWS_PER_S,
                                      ROWS_PER_S)])


_deg_call = functools.partial(
    pl.kernel,
    out_type=jax.ShapeDtypeStruct((2 * NPAD, DEGW), jnp.float32),
    mesh=_mesh,
    scratch_types=[
        pltpu.VMEM((CHUNK,), jnp.int32),
        pltpu.VMEM((CHUNK, DEGW), jnp.float32),
        pltpu.VMEM((ZR, DEGW), jnp.float32),
        pltpu.VMEM_SHARED((NPAD, DEGW), jnp.float32),
    ],
    compiler_params=pltpu.CompilerParams(use_tc_tiling_on_sc=False),
)(_deg_body)

_agg_call = functools.partial(
    pl.kernel,
    out_type=jax.ShapeDtypeStruct((2 * NPAD, D), jnp.float32),
    mesh=_mesh,
    scratch_types=[
        pltpu.VMEM((CHUNK,), jnp.int32),
        pltpu.VMEM((CHUNK,), jnp.int32),
        pltpu.VMEM((CHUNK, D), jnp.float32),
        pltpu.VMEM((ZR, D), jnp.float32),
        pltpu.VMEM_SHARED((NPAD, D), jnp.float32),
        pltpu.SemaphoreType.DMA,
    ],
    compiler_params=pltpu.CompilerParams(use_tc_tiling_on_sc=False),
)(_agg_body)


def _scale_body(state_ref, w_ref, degp_ref, g_ref, dinv_ref):
    h = jnp.dot(state_ref[...], w_ref[...], preferred_element_type=jnp.float32)
    deg = degp_ref[0][:N, 0:1] + degp_ref[1][:N, 0:1] + 1.0
    dinv = lax.rsqrt(deg)
    g_ref[...] = h * dinv
    dinv_ref[...] = dinv


def _head_body(accp_ref, g_ref, dinv_ref, state_ref, bconv_ref, w1_ref, b1_ref,
               w2_ref, b2_ref, w3_ref, b3_ref, out_ref):
    acc = accp_ref[0][:N] + accp_ref[1][:N] + g_ref[...]
    x = acc * dinv_ref[...] + bconv_ref[...]
    x = jnp.maximum(x, 0.0) + state_ref[...]
    y = jnp.dot(x, w1_ref[...], preferred_element_type=jnp.float32) + b1_ref[...]
    y = jnp.where(y > 0, y, 0.01 * y)
    z = jnp.dot(y, w2_ref[...], preferred_element_type=jnp.float32) + b2_ref[...]
    z = jnp.where(z > 0, z, 0.01 * z)
    c = jnp.dot(z, w3_ref[...], preferred_element_type=jnp.float32) + b3_ref[...]
    c = jnp.log1p(jnp.exp(-jnp.abs(c))) + jnp.maximum(c, 0.0)
    out_ref[...] = c / (jnp.sum(c) + 1e-20)


def kernel(state, edge_index, W_conv, b_conv, W1, b1, W2, b2, W3, b3,
           deterministic):
    src = edge_index[0]
    dst = edge_index[1]

    degp = _deg_call(dst)
    degp3 = degp.reshape(2, NPAD, DEGW)

    g, dinv = pl.pallas_call(
        _scale_body,
        out_shape=[
            jax.ShapeDtypeStruct((N, D), jnp.float32),
            jax.ShapeDtypeStruct((N, 1), jnp.float32),
        ],
    )(state, W_conv, degp3)

    accp = _agg_call(g, src, dst)
    accp3 = accp.reshape(2, NPAD, D)

    c = pl.pallas_call(
        _head_body,
        out_shape=jax.ShapeDtypeStruct((N, 1), jnp.float32),
    )(accp3, g, dinv, state, b_conv.reshape(1, D), W1, b1.reshape(1, H),
      W2, b2.reshape(1, H), W3, b3.reshape(1, 1))

    return c.reshape(N // A, A)


# speedup vs baseline: 28.1212x; 1.8180x over previous
import functools

import jax
import jax.numpy as jnp
from jax import lax
from jax.experimental import pallas as pl
from jax.experimental.pallas import tpu as pltpu
from jax.experimental.pallas import tpu_sc as plsc

N = 10000
E = 320000
D = 128
A = 8
H = 32

NC = 2
NS = 16
NW = NC * NS
CHUNK = 80
EDGES_PER_W = E // NW
NCHUNK = EDGES_PER_W // CHUNK
NPAD = 10240
DEGW = 16
ROWS_PER_S = NPAD // NS
ZR = 32
NBUF = 5
NBUF_A = 4
NGRP_A = NCHUNK // NBUF_A

_mesh = plsc.VectorSubcoreMesh(core_axis_name="c", subcore_axis_name="s")


def _deg_body(dst_hbm, degp_hbm, idx_v, ones_v, zer_v, deg_sh, isems, ssems):
    cid = lax.axis_index("c")
    sid = lax.axis_index("s")
    for r in range(CHUNK):
        ones_v[r, :] = jnp.full((16,), 1.0, jnp.float32)
    for r in range(ZR):
        zer_v[r, :] = jnp.zeros((16,), jnp.float32)
    for t in range(ROWS_PER_S // ZR):
        pltpu.sync_copy(zer_v, deg_sh.at[pl.ds(sid * ROWS_PER_S + t * ZR, ZR)])
    plsc.subcore_barrier()
    base = (cid * NS + sid) * EDGES_PER_W

    def body(j, carry):
        goff = base + j * (NBUF * CHUNK)
        idesc = []
        for b in range(NBUF):
            idesc.append(pltpu.async_copy(
                dst_hbm.at[pl.ds(goff + b * CHUNK, CHUNK)],
                idx_v.at[b], isems.at[b]))
        sdesc = []
        for b in range(NBUF):
            idesc[b].wait()
            sdesc.append(pltpu.async_copy(
                ones_v, deg_sh.at[idx_v.at[b]], ssems.at[b], add=True))
        for b in range(NBUF):
            sdesc[b].wait()
        return carry

    lax.fori_loop(0, NCHUNK // NBUF, body, 0)
    plsc.subcore_barrier()
    pltpu.sync_copy(deg_sh.at[pl.ds(sid * ROWS_PER_S, ROWS_PER_S)],
                    degp_hbm.at[pl.ds(cid * NPAD + sid * ROWS_PER_S,
                                      ROWS_PER_S)])


def _agg_body(g_hbm, src_hbm, dst_hbm, accp_hbm, src_v, dst_v, rows_v, zer_v,
              acc_sh, isems, jsems, gsems, ssems):
    cid = lax.axis_index("c")
    sid = lax.axis_index("s")
    for r in range(ZR):
        for k in range(D // 16):
            zer_v[r, pl.ds(k * 16, 16)] = jnp.zeros((16,), jnp.float32)
    for t in range(ROWS_PER_S // ZR):
        pltpu.sync_copy(zer_v, acc_sh.at[pl.ds(sid * ROWS_PER_S + t * ZR, ZR)])
    plsc.subcore_barrier()
    base = (cid * NS + sid) * EDGES_PER_W

    def group(goff, nb):
        idesc = []
        for b in range(nb):
            off = goff + b * CHUNK
            idesc.append((
                pltpu.async_copy(src_hbm.at[pl.ds(off, CHUNK)],
                                 src_v.at[b], isems.at[b]),
                pltpu.async_copy(dst_hbm.at[pl.ds(off, CHUNK)],
                                 dst_v.at[b], jsems.at[b]),
            ))
        gdesc = []
        for b in range(nb):
            idesc[b][0].wait()
            gdesc.append(pltpu.async_copy(
                g_hbm.at[src_v.at[b]], rows_v.at[b], gsems.at[b]))
        sdesc = []
        for b in range(nb):
            gdesc[b].wait()
            idesc[b][1].wait()
            sdesc.append(pltpu.async_copy(
                rows_v.at[b], acc_sh.at[dst_v.at[b]], ssems.at[b], add=True))
        for b in range(nb):
            sdesc[b].wait()

    def body(j, carry):
        group(base + j * (NBUF_A * CHUNK), NBUF_A)
        return carry

    lax.fori_loop(0, NGRP_A, body, 0)
    group(base + NGRP_A * (NBUF_A * CHUNK), NCHUNK - NGRP_A * NBUF_A)
    plsc.subcore_barrier()
    pltpu.sync_copy(acc_sh.at[pl.ds(sid * ROWS_PER_S, ROWS_PER_S)],
                    accp_hbm.at[pl.ds(cid * NPAD + sid * ROWS_PER_S,
                                      ROWS_PER_S)])


_deg_call = functools.partial(
    pl.kernel,
    out_type=jax.ShapeDtypeStruct((2 * NPAD, DEGW), jnp.float32),
    mesh=_mesh,
    scratch_types=[
        pltpu.VMEM((NBUF, CHUNK), jnp.int32),
        pltpu.VMEM((CHUNK, DEGW), jnp.float32),
        pltpu.VMEM((ZR, DEGW), jnp.float32),
        pltpu.VMEM_SHARED((NPAD, DEGW), jnp.float32),
        pltpu.SemaphoreType.DMA((NBUF,)),
        pltpu.SemaphoreType.DMA((NBUF,)),
    ],
    compiler_params=pltpu.CompilerParams(use_tc_tiling_on_sc=False),
)(_deg_body)

_agg_call = functools.partial(
    pl.kernel,
    out_type=jax.ShapeDtypeStruct((2 * NPAD, D), jnp.float32),
    mesh=_mesh,
    scratch_types=[
        pltpu.VMEM((NBUF_A, CHUNK), jnp.int32),
        pltpu.VMEM((NBUF_A, CHUNK), jnp.int32),
        pltpu.VMEM((NBUF_A, CHUNK, D), jnp.float32),
        pltpu.VMEM((ZR, D), jnp.float32),
        pltpu.VMEM_SHARED((NPAD, D), jnp.float32),
        pltpu.SemaphoreType.DMA((NBUF_A,)),
        pltpu.SemaphoreType.DMA((NBUF_A,)),
        pltpu.SemaphoreType.DMA((NBUF_A,)),
        pltpu.SemaphoreType.DMA((NBUF_A,)),
    ],
    compiler_params=pltpu.CompilerParams(use_tc_tiling_on_sc=False),
)(_agg_body)


def _scale_body(state_ref, w_ref, degp_ref, g_ref, dinv_ref):
    h = jnp.dot(state_ref[...], w_ref[...], preferred_element_type=jnp.float32)
    deg = degp_ref[0][:N, 0:1] + degp_ref[1][:N, 0:1] + 1.0
    dinv = lax.rsqrt(deg)
    g_ref[...] = h * dinv
    dinv_ref[...] = dinv


def _head_body(accp_ref, g_ref, dinv_ref, state_ref, bconv_ref, w1_ref, b1_ref,
               w2_ref, b2_ref, w3_ref, b3_ref, out_ref):
    acc = accp_ref[0][:N] + accp_ref[1][:N] + g_ref[...]
    x = acc * dinv_ref[...] + bconv_ref[...]
    x = jnp.maximum(x, 0.0) + state_ref[...]
    y = jnp.dot(x, w1_ref[...], preferred_element_type=jnp.float32) + b1_ref[...]
    y = jnp.where(y > 0, y, 0.01 * y)
    z = jnp.dot(y, w2_ref[...], preferred_element_type=jnp.float32) + b2_ref[...]
    z = jnp.where(z > 0, z, 0.01 * z)
    c = jnp.dot(z, w3_ref[...], preferred_element_type=jnp.float32) + b3_ref[...]
    c = jnp.log1p(jnp.exp(-jnp.abs(c))) + jnp.maximum(c, 0.0)
    out_ref[...] = c / (jnp.sum(c) + 1e-20)


def kernel(state, edge_index, W_conv, b_conv, W1, b1, W2, b2, W3, b3,
           deterministic):
    src = edge_index[0]
    dst = edge_index[1]

    degp = _deg_call(dst)
    degp3 = degp.reshape(2, NPAD, DEGW)

    g, dinv = pl.pallas_call(
        _scale_body,
        out_shape=[
            jax.ShapeDtypeStruct((N, D), jnp.float32),
            jax.ShapeDtypeStruct((N, 1), jnp.float32),
        ],
    )(state, W_conv, degp3)

    accp = _agg_call(g, src, dst)
    accp3 = accp.reshape(2, NPAD, D)

    c = pl.pallas_call(
        _head_body,
        out_shape=jax.ShapeDtypeStruct((N, 1), jnp.float32),
    )(accp3, g, dinv, state, b_conv.reshape(1, D), W1, b1.reshape(1, H),
      W2, b2.reshape(1, H), W3, b3.reshape(1, 1))

    return c.reshape(N // A, A)
